# baseline (speedup 1.0000x reference)
import functools

import jax
import jax.numpy as jnp
from jax import lax
from jax.experimental import pallas as pl
from jax.experimental.pallas import tpu as pltpu
from jax.experimental.pallas import tpu_sc as plsc

NX = 512
NY = 512
NCH = 64
NVOX = 20000

NC = 2
NS = 16
NW = NC * NS
RANGE = (NX * NY) // NW
RBITS = 13
SUBR = 512
NBIN = RANGE // SUBR

CHUNK = 2000
NCHUNK = NVOX // CHUNK
GPC = CHUNK // 16
K = 128
OWNCAP = NVOX + 2 * K


_GDN = lax.GatherDimensionNumbers(
    offset_dims=(), collapsed_slice_dims=(0,), start_index_map=(0,))


def _nextof(x, iota):
    idx = jnp.minimum(iota + 1, 15)
    return lax.gather(x, idx[:, None], _GDN, slice_sizes=(1,),
                      mode=lax.GatherScatterMode.PROMISE_IN_BOUNDS)


def _body(vf, c0_hbm, c1_hbm, out_hbm, slabA, slabB, owned, binbuf,
          c0b, c1b, gidx, featb, gsem, osemA, osemB):
    wid = lax.axis_index("s") * NC + lax.axis_index("c")
    lo = wid * RANGE
    iota = lax.broadcasted_iota(jnp.int32, (16,), 0)
    zeros16 = jnp.zeros((16,), jnp.float32)

    def chunk_body(g, cursor):
        pltpu.sync_copy(c0_hbm.at[pl.ds(g * CHUNK, CHUNK)], c0b)
        pltpu.sync_copy(c1_hbm.at[pl.ds(g * CHUNK, CHUNK)], c1b)

        def grp(i, cur):
            v0 = c0b[pl.ds(i * 16, 16)]
            v1 = c1b[pl.ds(i * 16, 16)]
            local = (v0 + v1 * NX) - lo
            m = (local >= 0) & (local < RANGE)
            vid = g * CHUNK + i * 16 + iota
            packed = (vid << RBITS) | jnp.where(m, local, 0)
            mi = m.astype(jnp.int32)
            pos = cur + plsc.cumsum(mi) - 1
            plsc.store_scatter(owned, [pos], packed, mask=m)
            return cur + jnp.sum(mi)

        return lax.fori_loop(0, GPC, grp, cursor)

    n = lax.fori_loop(0, NCHUNK, chunk_body, jnp.int32(0))
    ng = (n + 15) // 16

    def zrow(i, _):
        for c in range(NCH):
            slabA[c, pl.ds(i * 16, 16)] = zeros16
            slabB[c, pl.ds(i * 16, 16)] = zeros16
        return 0

    lax.fori_loop(0, SUBR // 16, zrow, 0)

    def subcompact(b):
        def g(j, cnt):
            pk = owned[pl.ds(j * 16, 16)]
            valid = (j * 16 + iota) < n
            local = pk & (RANGE - 1)
            mb = valid & ((local >> 9) == b)
            entry = ((pk >> RBITS) << 9) | (local & (SUBR - 1))
            mi = mb.astype(jnp.int32)
            pos = cnt + plsc.cumsum(mi) - 1
            plsc.store_scatter(binbuf, [pos], entry, mask=mb)
            return cnt + jnp.sum(mi)

        return lax.fori_loop(0, ng, g, jnp.int32(0))

    def clearbin(slab, nb):
        def g(j, _):
            e = binbuf[pl.ds(j * 16, 16)]
            mv = (j * 16 + iota) < nb
            col = e & (SUBR - 1)

            def cc(c8, _):
                for c in range(8):
                    cvec = iota * 0 + (c8 * 8 + c)
                    plsc.store_scatter(slab, [cvec, col], zeros16, mask=mv)
                return 0

            lax.fori_loop(0, NCH // 8, cc, 0)
            return 0

        lax.fori_loop(0, (nb + 15) // 16, g, 0)

    def scatterbin(slab, nb):
        def kchunk(k, _):
            base = k * K

            def gi(j, _):
                e = binbuf[pl.ds(base + j * 16, 16)]
                ok = (base + j * 16 + iota) < nb
                vid = jnp.minimum(e >> 9, NVOX - 1)
                gidx[pl.ds(j * 16, 16)] = jnp.where(ok, vid, j * 16 + iota)
                return 0

            lax.fori_loop(0, K // 16, gi, 0)
            pltpu.async_copy(vf.at[gidx], featb, gsem).wait()

            def sc(j, _):
                e = binbuf[pl.ds(base + j * 16, 16)]
                valid = (base + j * 16 + iota) < nb
                col = e & (SUBR - 1)
                key2 = (jnp.where(valid, col, SUBR + iota) << 4) | iota
                sk, sv = plsc.sort_key_val(key2, iota)
                skey = sk >> 4
                is_last = (_nextof(skey, iota) != skey) | (iota == 15)
                m = is_last & (skey < SUBR)
                row = j * 16 + sv

                def cc(c8, _):
                    for c in range(8):
                        cvec = iota * 0 + (c8 * 8 + c)
                        vals = plsc.load_gather(featb, [row, cvec])
                        plsc.store_scatter(slab, [cvec, skey], vals, mask=m)
                    return 0

                lax.fori_loop(0, NCH // 8, cc, 0)
                return 0

            lax.fori_loop(0, K // 16, sc, 0)
            return 0

        lax.fori_loop(0, (nb + K - 1) // K, kchunk, 0)

    slabs = (slabA, slabB)
    osems = (osemA, osemB)
    pend = [None, None]
    for b in range(NBIN):
        s = b & 1
        slab = slabs[s]
        if pend[s] is not None:
            pend[s].wait()
            clearbin(slab, subcompact(b - 2))
        nb = subcompact(b)
        scatterbin(slab, nb)
        pend[s] = pltpu.async_copy(
            slab, out_hbm.at[pl.ds(0, NCH), pl.ds(lo + b * SUBR, SUBR)],
            osems[s])
    for s in (0, 1):
        pend[s].wait()


@jax.jit
def kernel(voxel_features, coords):
    coords = coords.astype(jnp.int32)
    c0 = coords[:, 0]
    c1 = coords[:, 1]

    mesh = plsc.VectorSubcoreMesh(core_axis_name="c", subcore_axis_name="s")
    run = functools.partial(
        pl.kernel,
        out_type=jax.ShapeDtypeStruct((NCH, NX * NY), jnp.float32),
        mesh=mesh,
        compiler_params=pltpu.CompilerParams(
            needs_layout_passes=False, use_tc_tiling_on_sc=False),
        scratch_types=[
            pltpu.VMEM((NCH, SUBR), jnp.float32),
            pltpu.VMEM((NCH, SUBR), jnp.float32),
            pltpu.VMEM((OWNCAP,), jnp.int32),
            pltpu.VMEM((OWNCAP,), jnp.int32),
            pltpu.VMEM((CHUNK,), jnp.int32),
            pltpu.VMEM((CHUNK,), jnp.int32),
            pltpu.VMEM((K,), jnp.int32),
            pltpu.VMEM((K, NCH), jnp.float32),
            pltpu.SemaphoreType.DMA,
            pltpu.SemaphoreType.DMA,
            pltpu.SemaphoreType.DMA,
        ],
    )(_body)
    canvas = run(voxel_features, c0, c1)
    return canvas.reshape(NCH, NX, NY)


# speedup vs baseline: 1.2182x; 1.2182x over previous
import functools

import jax
import jax.numpy as jnp
from jax import lax
from jax.experimental import pallas as pl
from jax.experimental.pallas import tpu as pltpu
from jax.experimental.pallas import tpu_sc as plsc

NX = 512
NY = 512
NCH = 64
NVOX = 20000

NC = 2
NS = 16
NW = NC * NS
RANGE = (NX * NY) // NW
RBITS = 13

CPASS = 4
NPAIR = NCH // (2 * CPASS)
CHUNK = 2000
NCHUNK = NVOX // CHUNK
GPC = CHUNK // 16
K = 128
NQ = 8
SSZ = K * NQ
OWNCAP = NVOX + SSZ


_GDN = lax.GatherDimensionNumbers(
    offset_dims=(), collapsed_slice_dims=(0,), start_index_map=(0,))


def _vgather(x, idx):
    return lax.gather(x, idx[:, None], _GDN, slice_sizes=(1,),
                      mode=lax.GatherScatterMode.PROMISE_IN_BOUNDS)


def _body(vf8, c0_hbm, c1_hbm, out_hbm, slabA, slabB, owned, dpk,
          c0b, c1b, gq, featb, gsem, osemA, osemB):
    wid = lax.axis_index("s") * NC + lax.axis_index("c")
    lo = wid * RANGE
    iota = lax.broadcasted_iota(jnp.int32, (16,), 0)
    last15 = iota * 0 + 15
    zeros16 = jnp.zeros((16,), jnp.float32)

    def one_group(gbase, i, cur):
        v0 = c0b[pl.ds(i * 16, 16)]
        v1 = c1b[pl.ds(i * 16, 16)]
        local = (v0 + v1 * NX) - lo
        m = (local >= 0) & (local < RANGE)
        vid = gbase + i * 16 + iota
        packed = (vid << RBITS) | jnp.where(m, local, 0)
        s = plsc.cumsum(m.astype(jnp.int32))
        plsc.store_scatter(owned, [cur + s - 1], packed, mask=m)
        return cur + _vgather(s, last15)

    def chunk_body(g, cur):
        pltpu.sync_copy(c0_hbm.at[pl.ds(g * CHUNK, CHUNK)], c0b)
        pltpu.sync_copy(c1_hbm.at[pl.ds(g * CHUNK, CHUNK)], c1b)

        def grp(i, cur):
            cur = one_group(g * CHUNK, 2 * i, cur)
            return one_group(g * CHUNK, 2 * i + 1, cur)

        cur = lax.fori_loop(0, GPC // 2, grp, cur)
        return one_group(g * CHUNK, GPC - 1, cur)

    nvec = lax.fori_loop(0, NCHUNK, chunk_body, iota * 0)
    n = jnp.max(nvec)
    ng = (n + 15) // 16

    def dedup(j, _):
        pk = owned[pl.ds(j * 16, 16)]
        valid = (j * 16 + iota) < n
        local = pk & (RANGE - 1)
        key2 = (jnp.where(valid, local, RANGE + iota) << 4) | iota
        sk, sv = plsc.sort_key_val(key2, iota)
        skey = sk >> 4
        nxt = _vgather(skey, jnp.minimum(iota + 1, 15))
        m = ((nxt != skey) | (iota == 15)) & (skey < RANGE)
        dpk[pl.ds(j * 16, 16)] = (skey << 5) | (sv << 1) | m.astype(jnp.int32)
        return 0

    lax.fori_loop(0, ng, dedup, 0)

    def zrow(i, _):
        for c in range(CPASS):
            slabA[c, pl.ds(i * 16, 16)] = zeros16
            slabB[c, pl.ds(i * 16, 16)] = zeros16
        return 0

    lax.fori_loop(0, RANGE // 16, zrow, 0)

    nss = (n + SSZ - 1) // SSZ

    def run_pass(q):
        def superstep(ss, _):
            sbase = ss * SSZ

            dmas = []
            for t in range(NQ):
                qbase = sbase + t * K

                def gi(j, _, qbase=qbase, t=t):
                    pk = owned[pl.ds(qbase + j * 16, 16)]
                    ok = (qbase + j * 16 + iota) < n
                    gq[t, pl.ds(j * 16, 16)] = jnp.where(
                        ok, (pk >> RBITS) * NPAIR + q, j * 16 + iota)
                    return 0

                lax.fori_loop(0, K // 16, gi, 0)
                dmas.append(pltpu.async_copy(
                    vf8.at[gq.at[t]], featb.at[pl.ds(t * K, K)], gsem))
            for d in dmas:
                d.wait()

            def sc(j, _):
                dp = dpk[pl.ds(sbase + j * 16, 16)]
                m = (dp & 1) == 1
                skey = dp >> 5
                sv = (dp >> 1) & 15
                row = j * 16 + sv
                for c in range(CPASS):
                    cvec = iota * 0 + c
                    vals = plsc.load_gather(featb, [row, cvec])
                    plsc.store_scatter(slabA, [cvec, skey], vals, mask=m)
                    vals = plsc.load_gather(featb, [row, cvec + CPASS])
                    plsc.store_scatter(slabB, [cvec, skey], vals, mask=m)
                return 0

            strip = jnp.clip((n - sbase + 15) >> 4, 0, SSZ // 16)
            lax.fori_loop(0, strip, sc, 0)
            return 0

        lax.fori_loop(0, nss, superstep, 0)
        da = pltpu.async_copy(
            slabA,
            out_hbm.at[pl.ds(q * 2 * CPASS, CPASS), pl.ds(lo, RANGE)], osemA)
        db = pltpu.async_copy(
            slabB,
            out_hbm.at[pl.ds(q * 2 * CPASS + CPASS, CPASS), pl.ds(lo, RANGE)],
            osemB)
        return da, db

    pend = None
    for q in range(NPAIR):
        if pend is not None:
            pend[0].wait()
            pend[1].wait()
        pend = run_pass(q)
    pend[0].wait()
    pend[1].wait()


@jax.jit
def kernel(voxel_features, coords):
    coords = coords.astype(jnp.int32)
    c0 = coords[:, 0]
    c1 = coords[:, 1]
    vf8 = voxel_features.reshape(NVOX * NPAIR, 2 * CPASS)

    mesh = plsc.VectorSubcoreMesh(core_axis_name="c", subcore_axis_name="s")
    run = functools.partial(
        pl.kernel,
        out_type=jax.ShapeDtypeStruct((NCH, NX * NY), jnp.float32),
        mesh=mesh,
        compiler_params=pltpu.CompilerParams(
            needs_layout_passes=False, use_tc_tiling_on_sc=False),
        scratch_types=[
            pltpu.VMEM((CPASS, RANGE), jnp.float32),
            pltpu.VMEM((CPASS, RANGE), jnp.float32),
            pltpu.VMEM((OWNCAP,), jnp.int32),
            pltpu.VMEM((OWNCAP,), jnp.int32),
            pltpu.VMEM((CHUNK,), jnp.int32),
            pltpu.VMEM((CHUNK,), jnp.int32),
            pltpu.VMEM((NQ, K), jnp.int32),
            pltpu.VMEM((SSZ, 2 * CPASS), jnp.float32),
            pltpu.SemaphoreType.DMA,
            pltpu.SemaphoreType.DMA,
            pltpu.SemaphoreType.DMA,
        ],
    )(_body)
    canvas = run(vf8, c0, c1)
    return canvas.reshape(NCH, NX, NY)
